# baseline (speedup 1.0000x reference)
import functools

import jax
import jax.numpy as jnp
from jax import lax
from jax.experimental import pallas as pl
from jax.experimental.pallas import tpu as pltpu
from jax.experimental.pallas import tpu_sc as plsc

N = 10000
E = 320000
D = 128
NC = 2
NS = 16
NW = NC * NS
EPW = E // NW
B = 128
K = -(-EPW // B)
PAD = K * B - EPW
RPT = 632
NR = RPT * NS
KC = 8
CHUNKS = (128, 128, 128, 128, 120)

_mesh = plsc.VectorSubcoreMesh(
    core_axis_name="c", subcore_axis_name="s", num_cores=NC, num_subcores=NS)


def _fill_tile_buf(ref, nrows, width, value):
  v = jnp.full((16,), value, jnp.float32)

  def row(i, c):
    for cc in range(width // 16):
      ref[i, pl.ds(cc * 16, 16)] = v
    return c

  lax.fori_loop(0, nrows, row, 0)


def _stage_zero(zbuf, shared, r0):
  off = 0
  for sz in CHUNKS:
    pltpu.sync_copy(zbuf.at[pl.ds(0, sz)],
                    shared.at[pl.ds(pl.multiple_of(r0 + off, 8), sz)])
    off += sz


def _stage_out(shared, buf, r0, out_hbm, obase):
  off = 0
  for sz in CHUNKS:
    pltpu.sync_copy(shared.at[pl.ds(pl.multiple_of(r0 + off, 8), sz)],
                    buf.at[pl.ds(0, sz)])
    pltpu.sync_copy(buf.at[pl.ds(0, sz)],
                    out_hbm.at[pl.ds(pl.multiple_of(obase + off, 8), sz)])
    off += sz


def _edge_loop(body_fn, src_hbm, dst_hbm, src_v, dst_v, w):
  def outer(jj, carry):
    e0 = pl.multiple_of(w * K * NS + jj * KC, 8)
    pltpu.sync_copy(src_hbm.at[pl.ds(e0, KC)], src_v)
    pltpu.sync_copy(dst_hbm.at[pl.ds(e0, KC)], dst_v)

    def step(j, c):
      body_fn(j)
      return c

    lax.fori_loop(0, KC, step, 0)
    return carry

  lax.fori_loop(0, K * NS // KC, outer, 0)


def _sc_agg_cnt_body(x_hbm, src_hbm, dst_hbm, agg_out, cnt_out, src_v, dst_v,
                     rows_v, acc_sh, sem):
  cid = lax.axis_index("c")
  sid = lax.axis_index("s")
  w = cid * NS + sid
  r0 = pl.multiple_of(sid * RPT, 8)
  obase = pl.multiple_of(cid * NR + r0, 8)
  _fill_tile_buf(rows_v, B, D, 0.0)
  _stage_zero(rows_v, acc_sh, r0)
  plsc.subcore_barrier()

  def step1(j):
    pltpu.async_copy(x_hbm.at[src_v.at[j]], rows_v, sem).wait()
    pltpu.sync_copy(rows_v, acc_sh.at[dst_v.at[j]], add=True)

  @pl.when(sid == 0)
  def _():
    _edge_loop(step1, src_hbm, dst_hbm, src_v, dst_v, cid)
  plsc.subcore_barrier()
  _stage_out(acc_sh, rows_v, r0, agg_out, obase)
  _fill_tile_buf(rows_v, B, D, 0.0)
  _stage_zero(rows_v, acc_sh, r0)
  _fill_tile_buf(rows_v, B, D, 1.0)
  plsc.subcore_barrier()

  def step2(j):
    e0 = pl.multiple_of(w * K, 8)
    del e0
    pltpu.sync_copy(rows_v, acc_sh.at[dst_v.at[j]], add=True)

  @pl.when(sid == 0)
  def _():
    _edge_loop(step2, src_hbm, dst_hbm, src_v, dst_v, cid)
  plsc.subcore_barrier()
  _stage_out(acc_sh, rows_v, r0, cnt_out, obase)


def _sc_agg_body(x_hbm, src_hbm, dst_hbm, agg_out, src_v, dst_v, rows_v,
                 acc_sh, sem):
  cid = lax.axis_index("c")
  sid = lax.axis_index("s")
  w = cid * NS + sid
  r0 = pl.multiple_of(sid * RPT, 8)
  obase = pl.multiple_of(cid * NR + r0, 8)
  _fill_tile_buf(rows_v, B, D, 0.0)
  _stage_zero(rows_v, acc_sh, r0)
  plsc.subcore_barrier()

  def step1(j):
    pltpu.async_copy(x_hbm.at[src_v.at[j]], rows_v, sem).wait()
    pltpu.sync_copy(rows_v, acc_sh.at[dst_v.at[j]], add=True)

  @pl.when(sid == 0)
  def _():
    _edge_loop(step1, src_hbm, dst_hbm, src_v, dst_v, cid)
  plsc.subcore_barrier()
  _stage_out(acc_sh, rows_v, r0, agg_out, obase)


_sc_scratch = [
    pltpu.VMEM((KC, B), jnp.int32),
    pltpu.VMEM((KC, B), jnp.int32),
    pltpu.VMEM((B, D), jnp.float32),
    pltpu.VMEM_SHARED((NR, D), jnp.float32),
    pltpu.SemaphoreType.DMA,
]

_sc_agg_cnt = pl.kernel(
    _sc_agg_cnt_body,
    out_type=(jax.ShapeDtypeStruct((NC * NR, D), jnp.float32),
              jax.ShapeDtypeStruct((NC * NR, D), jnp.float32)),
    mesh=_mesh,
    scratch_types=list(_sc_scratch))

_sc_agg = pl.kernel(
    _sc_agg_body,
    out_type=(jax.ShapeDtypeStruct((NC * NR, D), jnp.float32),),
    mesh=_mesh,
    scratch_types=list(_sc_scratch))

BM = 1000


def _tc_layer_body(relu, agg_ref, cnt_ref, x_ref, wl_ref, wr_ref, b_ref,
                   o_ref):
  a = agg_ref[0] + agg_ref[1]
  c = cnt_ref[0][:, 0:1] + cnt_ref[1][:, 0:1]
  mean = a / jnp.maximum(c, 1.0)
  r = (jnp.dot(mean, wl_ref[...], preferred_element_type=jnp.float32,
               precision=lax.Precision.HIGHEST)
       + jnp.dot(x_ref[...], wr_ref[...], preferred_element_type=jnp.float32,
                 precision=lax.Precision.HIGHEST)
       + b_ref[...])
  o_ref[...] = jnp.maximum(r, 0.0) if relu else r


def _tc_layer(agg, cnt, x, wl, wr, b, relu):
  return pl.pallas_call(
      functools.partial(_tc_layer_body, relu),
      grid=(N // BM,),
      in_specs=[
          pl.BlockSpec((NC, BM, D), lambda i: (0, i, 0)),
          pl.BlockSpec((NC, BM, D), lambda i: (0, i, 0)),
          pl.BlockSpec((BM, D), lambda i: (i, 0)),
          pl.BlockSpec((D, D), lambda i: (0, 0)),
          pl.BlockSpec((D, D), lambda i: (0, 0)),
          pl.BlockSpec((1, D), lambda i: (0, 0)),
      ],
      out_specs=pl.BlockSpec((BM, D), lambda i: (i, 0)),
      out_shape=jax.ShapeDtypeStruct((N, D), jnp.float32),
  )(agg, cnt, x, wl, wr, b)


def kernel(x, edge_index, Wl1, Wr1, b1, Wl2, Wr2, b2):
  src = edge_index[0].astype(jnp.int32).reshape(NW, EPW)
  dst = edge_index[1].astype(jnp.int32).reshape(NW, EPW)
  src2 = jnp.pad(src, ((0, 0), (0, PAD))).reshape(NW * K, B)
  dst2 = jnp.pad(dst, ((0, 0), (0, PAD)),
                 constant_values=N).reshape(NW * K, B)
  b1r = b1.reshape(1, D)
  b2r = b2.reshape(1, D)

  agg1, cnt = _sc_agg_cnt(x, src2, dst2)
  agg1 = agg1.reshape(NC, NR, D)
  cnt = cnt.reshape(NC, NR, D)
  h = _tc_layer(agg1, cnt, x, Wl1, Wr1, b1r, relu=True)
  (agg2,) = _sc_agg(h, src2, dst2)
  out = _tc_layer(agg2.reshape(NC, NR, D), cnt, h, Wl2, Wr2, b2r, relu=False)
  return out


# speedup vs baseline: 1.0217x; 1.0217x over previous
import functools

import jax
import jax.numpy as jnp
from jax import lax
from jax.experimental import pallas as pl
from jax.experimental.pallas import tpu as pltpu
from jax.experimental.pallas import tpu_sc as plsc

N = 10000
E = 320000
D = 128
NC = 2
NS = 16
NW = NC * NS
EPW = E // NW
B = 128
K = -(-EPW // B)
PAD = K * B - EPW
RPT = 632
NR = RPT * NS
KC = 8
CHUNKS = (128, 128, 128, 128, 120)

_mesh = plsc.VectorSubcoreMesh(
    core_axis_name="c", subcore_axis_name="s", num_cores=NC, num_subcores=NS)


def _fill_tile_buf(ref, nrows, width, value):
  v = jnp.full((16,), value, jnp.float32)

  def row(i, c):
    for cc in range(width // 16):
      ref[i, pl.ds(cc * 16, 16)] = v
    return c

  lax.fori_loop(0, nrows, row, 0)


def _stage_zero(zbuf, shared, r0):
  off = 0
  for sz in CHUNKS:
    pltpu.sync_copy(zbuf.at[pl.ds(0, sz)],
                    shared.at[pl.ds(pl.multiple_of(r0 + off, 8), sz)])
    off += sz


def _stage_out(shared, buf, r0, out_hbm, obase):
  off = 0
  for sz in CHUNKS:
    pltpu.sync_copy(shared.at[pl.ds(pl.multiple_of(r0 + off, 8), sz)],
                    buf.at[pl.ds(0, sz)])
    pltpu.sync_copy(buf.at[pl.ds(0, sz)],
                    out_hbm.at[pl.ds(pl.multiple_of(obase + off, 8), sz)])
    off += sz


def _edge_loop(body_fn, src_hbm, dst_hbm, src_v, dst_v, cid, gather):
  def outer(jj, carry):
    e0 = pl.multiple_of(cid * K * NS + jj * KC, 8)
    if gather:
      pltpu.sync_copy(src_hbm.at[pl.ds(e0, KC)], src_v)
    pltpu.sync_copy(dst_hbm.at[pl.ds(e0, KC)], dst_v)

    def step(j, c):
      body_fn(j)
      return c

    lax.fori_loop(0, KC, step, 0)
    return carry

  lax.fori_loop(0, K * NS // KC, outer, 0)


def _sc_agg_cnt_body(x_hbm, src_hbm, dst_hbm, agg_out, cnt_out, src_v, dst_v,
                     rows_v, acc_sh, sem):
  cid = lax.axis_index("c")
  sid = lax.axis_index("s")
  r0 = pl.multiple_of(sid * RPT, 8)
  obase = pl.multiple_of(cid * NR + r0, 8)
  _fill_tile_buf(rows_v, B, D, 0.0)
  _stage_zero(rows_v, acc_sh, r0)
  plsc.subcore_barrier()

  def step1(j):
    pltpu.async_copy(x_hbm.at[src_v.at[j]], rows_v, sem).wait()
    pltpu.sync_copy(rows_v, acc_sh.at[dst_v.at[j]], add=True)

  @pl.when(sid == 0)
  def _():
    _edge_loop(step1, src_hbm, dst_hbm, src_v, dst_v, cid, gather=True)
  plsc.subcore_barrier()
  _stage_out(acc_sh, rows_v, r0, agg_out, obase)
  _fill_tile_buf(rows_v, B, D, 0.0)
  _stage_zero(rows_v, acc_sh, r0)
  _fill_tile_buf(rows_v, B, D, 1.0)
  plsc.subcore_barrier()

  def step2(j):
    pltpu.sync_copy(rows_v, acc_sh.at[dst_v.at[j]], add=True)

  @pl.when(sid == 0)
  def _():
    _edge_loop(step2, src_hbm, dst_hbm, src_v, dst_v, cid, gather=False)
  plsc.subcore_barrier()
  _stage_out(acc_sh, rows_v, r0, cnt_out, obase)


def _sc_agg_body(x_hbm, src_hbm, dst_hbm, agg_out, src_v, dst_v, rows_v,
                 acc_sh, sem):
  cid = lax.axis_index("c")
  sid = lax.axis_index("s")
  r0 = pl.multiple_of(sid * RPT, 8)
  obase = pl.multiple_of(cid * NR + r0, 8)
  _fill_tile_buf(rows_v, B, D, 0.0)
  _stage_zero(rows_v, acc_sh, r0)
  plsc.subcore_barrier()

  def step1(j):
    pltpu.async_copy(x_hbm.at[src_v.at[j]], rows_v, sem).wait()
    pltpu.sync_copy(rows_v, acc_sh.at[dst_v.at[j]], add=True)

  @pl.when(sid == 0)
  def _():
    _edge_loop(step1, src_hbm, dst_hbm, src_v, dst_v, cid, gather=True)
  plsc.subcore_barrier()
  _stage_out(acc_sh, rows_v, r0, agg_out, obase)


_sc_scratch = [
    pltpu.VMEM((KC, B), jnp.int32),
    pltpu.VMEM((KC, B), jnp.int32),
    pltpu.VMEM((B, D), jnp.float32),
    pltpu.VMEM_SHARED((NR, D), jnp.float32),
    pltpu.SemaphoreType.DMA,
]

_sc_agg_cnt = pl.kernel(
    _sc_agg_cnt_body,
    out_type=(jax.ShapeDtypeStruct((NC * NR, D), jnp.float32),
              jax.ShapeDtypeStruct((NC * NR, D), jnp.float32)),
    mesh=_mesh,
    scratch_types=list(_sc_scratch))

_sc_agg = pl.kernel(
    _sc_agg_body,
    out_type=(jax.ShapeDtypeStruct((NC * NR, D), jnp.float32),),
    mesh=_mesh,
    scratch_types=list(_sc_scratch))

BM = 1000


def _tc_layer_body(relu, agg_ref, cnt_ref, x_ref, wl_ref, wr_ref, b_ref,
                   o_ref):
  a = agg_ref[0] + agg_ref[1]
  c = cnt_ref[0][:, 0:1] + cnt_ref[1][:, 0:1]
  mean = a / jnp.maximum(c, 1.0)
  r = (jnp.dot(mean, wl_ref[...], preferred_element_type=jnp.float32,
               precision=lax.Precision.HIGHEST)
       + jnp.dot(x_ref[...], wr_ref[...], preferred_element_type=jnp.float32,
                 precision=lax.Precision.HIGHEST)
       + b_ref[...])
  o_ref[...] = jnp.maximum(r, 0.0) if relu else r


def _tc_layer(agg, cnt, x, wl, wr, b, relu):
  return pl.pallas_call(
      functools.partial(_tc_layer_body, relu),
      grid=(N // BM,),
      in_specs=[
          pl.BlockSpec((NC, BM, D), lambda i: (0, i, 0)),
          pl.BlockSpec((NC, BM, D), lambda i: (0, i, 0)),
          pl.BlockSpec((BM, D), lambda i: (i, 0)),
          pl.BlockSpec((D, D), lambda i: (0, 0)),
          pl.BlockSpec((D, D), lambda i: (0, 0)),
          pl.BlockSpec((1, D), lambda i: (0, 0)),
      ],
      out_specs=pl.BlockSpec((BM, D), lambda i: (i, 0)),
      out_shape=jax.ShapeDtypeStruct((N, D), jnp.float32),
  )(agg, cnt, x, wl, wr, b)


def kernel(x, edge_index, Wl1, Wr1, b1, Wl2, Wr2, b2):
  src = edge_index[0].astype(jnp.int32).reshape(NW, EPW)
  dst = edge_index[1].astype(jnp.int32).reshape(NW, EPW)
  src2 = jnp.pad(src, ((0, 0), (0, PAD))).reshape(NW * K, B)
  dst2 = jnp.pad(dst, ((0, 0), (0, PAD)),
                 constant_values=N).reshape(NW * K, B)
  b1r = b1.reshape(1, D)
  b2r = b2.reshape(1, D)

  agg1, cnt = _sc_agg_cnt(x, src2, dst2)
  agg1 = agg1.reshape(NC, NR, D)
  cnt = cnt.reshape(NC, NR, D)
  h = _tc_layer(agg1, cnt, x, Wl1, Wr1, b1r, relu=True)
  (agg2,) = _sc_agg(h, src2, dst2)
  out = _tc_layer(agg2.reshape(NC, NR, D), cnt, h, Wl2, Wr2, b2r, relu=False)
  return out
